# baseline (speedup 1.0000x reference)
import functools

import jax
import jax.numpy as jnp
from jax import lax
from jax.experimental import pallas as pl
from jax.experimental.pallas import tpu as pltpu
from jax.experimental.pallas import tpu_sc as plsc

N_NODES = 100000
N_EDGES = 3200000
D_EDGE = 16
D_NODE = 128
D_OUT = 128

NC = 2
NS = 16
LANES = 16
FB = 8
EB = N_EDGES // 128
EBC = EB // NC

CB = 25
CHUNK = CB * 128
NT = EBC // CB
NTPAIR = NT // 2
UNROLL = 25
N_COLS = 100352


def _sc_scatter_t(xe4, ei4):
    mesh = plsc.VectorSubcoreMesh(core_axis_name="c", subcore_axis_name="s")

    @functools.partial(
        pl.kernel,
        out_type=(
            jax.ShapeDtypeStruct((NC * NS, N_COLS), jnp.float32),
            jax.ShapeDtypeStruct((NC, EB, FB, 128), jnp.float32),
            jax.ShapeDtypeStruct((EB, 2, 128), jnp.int32),
        ),
        mesh=mesh,
        compiler_params=pltpu.CompilerParams(
            use_tc_tiling_on_sc=False, needs_layout_passes=False),
        scratch_types=[
            pltpu.VMEM((N_COLS,), jnp.float32),
            pltpu.VMEM((CB, 128), jnp.float32),
            pltpu.VMEM((CB, 128), jnp.float32),
            pltpu.VMEM((CB, 2, 128), jnp.int32),
            pltpu.VMEM((CB, 2, 128), jnp.int32),
            pltpu.SemaphoreType.DMA,
            pltpu.SemaphoreType.DMA,
            pltpu.SemaphoreType.DMA,
            pltpu.SemaphoreType.DMA,
        ],
    )
    def k(xe_hbm, ei_hbm, out_hbm, oxe_hbm, oei_hbm, acc,
          xb0, xb1, ib0, ib1, sl0, sl1, se0, se1):
        c = lax.axis_index("c")
        s = lax.axis_index("s")
        fb = s // FB
        fr = s % FB

        xb = (xb0, xb1)
        ib = (ib0, ib1)
        sl = (sl0, sl1)
        se = (se0, se1)

        zero = jnp.zeros((LANES,), jnp.float32)

        def zb(i, carry):
            acc[pl.ds(i * (4 * LANES), LANES)] = zero
            acc[pl.ds(i * (4 * LANES) + LANES, LANES)] = zero
            acc[pl.ds(i * (4 * LANES) + 2 * LANES, LANES)] = zero
            acc[pl.ds(i * (4 * LANES) + 3 * LANES, LANES)] = zero
            return carry

        lax.fori_loop(0, N_COLS // (4 * LANES), zb, 0)

        jcore = c * EBC

        def issue_loads(t, b):
            jbase = jcore + t * CB
            pltpu.async_copy(
                xe_hbm.at[fb, pl.ds(jbase, CB), fr, :], xb[b], sl[b])
            pltpu.async_copy(ei_hbm.at[pl.ds(jbase, CB)], ib[b], sl[b])

        def drain_loads(b):
            pltpu.make_async_copy(
                xe_hbm.at[0, pl.ds(0, CB), 0, :], xb[b], sl[b]).wait()
            pltpu.make_async_copy(
                ei_hbm.at[pl.ds(0, CB)], ib[b], sl[b]).wait()

        def issue_echo(t, b):
            jbase = jcore + t * CB
            pltpu.async_copy(
                xb[b], oxe_hbm.at[fb, pl.ds(jbase, CB), fr, :], se[b])

            @pl.when(s == 0)
            def _():
                pltpu.async_copy(ib[b], oei_hbm.at[pl.ds(jbase, CB)], se[b])

        def drain_echo(b):
            pltpu.make_async_copy(
                xb[b], oxe_hbm.at[0, pl.ds(0, CB), 0, :], se[b]).wait()

            @pl.when(s == 0)
            def _():
                pltpu.make_async_copy(
                    ib[b], oei_hbm.at[pl.ds(0, CB)], se[b]).wait()

        def compute(b):
            @plsc.parallel_loop(0, CB, 1, unroll=UNROLL)
            def grp(jj):
                for u in range(FB):
                    sl16 = pl.ds(u * LANES, LANES)
                    v = xb[b][jj, sl16]
                    i0 = ib[b][jj, 0, sl16]
                    i1 = ib[b][jj, 1, sl16]
                    plsc.addupdate_scatter(acc, [i0], v)
                    plsc.addupdate_scatter(acc, [i1], v)

        issue_loads(0, 0)

        def pair(i, carry):
            t0 = 2 * i
            drain_loads(0)
            issue_echo(t0, 0)

            @pl.when(i > 0)
            def _():
                drain_echo(1)

            issue_loads(t0 + 1, 1)
            compute(0)
            drain_loads(1)
            issue_echo(t0 + 1, 1)
            drain_echo(0)

            @pl.when(i < NTPAIR - 1)
            def _():
                issue_loads(t0 + 2, 0)

            compute(1)
            return carry

        lax.fori_loop(0, NTPAIR, pair, 0)
        drain_echo(1)

        pltpu.sync_copy(acc, out_hbm.at[c * NS + s])

    return k(xe4, ei4)


RB = 1024


def _tc_mlp(x_node, agg_t, W, b2):
    def mm(x_ref, p_ref, w_ref, b_ref, o_ref):
        w = w_ref[...]
        p = p_ref[...]
        a_t = p[:NS, :] + p[NS:, :]
        acc = jnp.dot(x_ref[...], w[:D_NODE, :], preferred_element_type=jnp.float32)
        acc += lax.dot_general(
            a_t, w[D_NODE:, :],
            dimension_numbers=(((0,), (0,)), ((), ())),
            preferred_element_type=jnp.float32,
        )
        o_ref[...] = acc + b_ref[...]

    nb = N_COLS // RB
    return pl.pallas_call(
        mm,
        grid=(nb,),
        in_specs=[
            pl.BlockSpec((RB, D_NODE), lambda i: (i, 0)),
            pl.BlockSpec((NC * NS, RB), lambda i: (0, i)),
            pl.BlockSpec((D_NODE + D_EDGE, D_OUT), lambda i: (0, 0)),
            pl.BlockSpec((1, D_OUT), lambda i: (0, 0)),
        ],
        out_specs=pl.BlockSpec((RB, D_OUT), lambda i: (i, 0)),
        out_shape=jax.ShapeDtypeStruct((N_NODES, D_OUT), jnp.float32),
    )(x_node, agg_t, W, b2)


def kernel(x_node, x_edge, edge_index, W, b):
    xe4 = x_edge.T.reshape(NC, FB, EB, 128).transpose(0, 2, 1, 3)
    ei4 = edge_index.T.reshape(2, EB, 128).transpose(1, 0, 2)
    agg_t, xe4_out, ei4_out = _sc_scatter_t(xe4, ei4)
    out = _tc_mlp(x_node, agg_t, W, b.reshape(1, D_OUT))
    x_edge_out = xe4_out.transpose(0, 2, 1, 3).reshape(D_EDGE, N_EDGES).T
    edge_index_out = ei4_out.transpose(1, 0, 2).reshape(2, N_EDGES).T
    return (out, x_edge_out, edge_index_out)


# speedup vs baseline: 1.2764x; 1.2764x over previous
import functools

import jax
import jax.numpy as jnp
from jax import lax
from jax.experimental import pallas as pl
from jax.experimental.pallas import tpu as pltpu
from jax.experimental.pallas import tpu_sc as plsc

N_NODES = 100000
N_EDGES = 3200000
D_EDGE = 16
D_NODE = 128
D_OUT = 128

NC = 2
NS = 16
LANES = 16
FB = 8
EB = N_EDGES // 128
EBC = EB // NC

CB = 25
CHUNK = CB * 128
NT = EBC // CB
NTPAIR = NT // 2
UNROLL = 5
N_COLS = 100352


def _sc_scatter_t(xe4, ei4):
    mesh = plsc.VectorSubcoreMesh(core_axis_name="c", subcore_axis_name="s")

    @functools.partial(
        pl.kernel,
        out_type=(
            jax.ShapeDtypeStruct((NC * NS, N_COLS), jnp.float32),
            jax.ShapeDtypeStruct((NC, EB, FB, 128), jnp.float32),
            jax.ShapeDtypeStruct((EB, 2, 128), jnp.int32),
        ),
        mesh=mesh,
        compiler_params=pltpu.CompilerParams(
            use_tc_tiling_on_sc=False, needs_layout_passes=False),
        scratch_types=[
            pltpu.VMEM((N_COLS,), jnp.float32),
            pltpu.VMEM((CB, 128), jnp.float32),
            pltpu.VMEM((CB, 128), jnp.float32),
            pltpu.VMEM((CB, 2, 128), jnp.int32),
            pltpu.VMEM((CB, 2, 128), jnp.int32),
            pltpu.SemaphoreType.DMA,
            pltpu.SemaphoreType.DMA,
            pltpu.SemaphoreType.DMA,
            pltpu.SemaphoreType.DMA,
        ],
    )
    def k(xe_hbm, ei_hbm, out_hbm, oxe_hbm, oei_hbm, acc,
          xb0, xb1, ib0, ib1, sl0, sl1, se0, se1):
        c = lax.axis_index("c")
        s = lax.axis_index("s")
        fb = s // FB
        fr = s % FB

        xb = (xb0, xb1)
        ib = (ib0, ib1)
        sl = (sl0, sl1)
        se = (se0, se1)

        zero = jnp.zeros((LANES,), jnp.float32)

        def zb(i, carry):
            acc[pl.ds(i * (4 * LANES), LANES)] = zero
            acc[pl.ds(i * (4 * LANES) + LANES, LANES)] = zero
            acc[pl.ds(i * (4 * LANES) + 2 * LANES, LANES)] = zero
            acc[pl.ds(i * (4 * LANES) + 3 * LANES, LANES)] = zero
            return carry

        lax.fori_loop(0, N_COLS // (4 * LANES), zb, 0)

        jcore = c * EBC

        def issue_loads(t, b):
            jbase = jcore + t * CB
            pltpu.async_copy(
                xe_hbm.at[fb, pl.ds(jbase, CB), fr, :], xb[b], sl[b])
            pltpu.async_copy(ei_hbm.at[pl.ds(jbase, CB)], ib[b], sl[b])

        def drain_loads(b):
            pltpu.make_async_copy(
                xe_hbm.at[0, pl.ds(0, CB), 0, :], xb[b], sl[b]).wait()
            pltpu.make_async_copy(
                ei_hbm.at[pl.ds(0, CB)], ib[b], sl[b]).wait()

        def issue_echo(t, b):
            jbase = jcore + t * CB
            pltpu.async_copy(
                xb[b], oxe_hbm.at[fb, pl.ds(jbase, CB), fr, :], se[b])

            @pl.when(s == 0)
            def _():
                pltpu.async_copy(ib[b], oei_hbm.at[pl.ds(jbase, CB)], se[b])

        def drain_echo(b):
            pltpu.make_async_copy(
                xb[b], oxe_hbm.at[0, pl.ds(0, CB), 0, :], se[b]).wait()

            @pl.when(s == 0)
            def _():
                pltpu.make_async_copy(
                    ib[b], oei_hbm.at[pl.ds(0, CB)], se[b]).wait()

        def compute(b):
            @plsc.parallel_loop(0, CB, 1, unroll=UNROLL)
            def grp(jj):
                for u in range(FB):
                    sl16 = pl.ds(u * LANES, LANES)
                    v = xb[b][jj, sl16]
                    i0 = ib[b][jj, 0, sl16]
                    i1 = ib[b][jj, 1, sl16]
                    plsc.addupdate_scatter(acc, [i0], v)
                    plsc.addupdate_scatter(acc, [i1], v)

        issue_loads(0, 0)

        def pair(i, carry):
            t0 = 2 * i
            drain_loads(0)
            issue_echo(t0, 0)

            @pl.when(i > 0)
            def _():
                drain_echo(1)

            issue_loads(t0 + 1, 1)
            compute(0)
            drain_loads(1)
            issue_echo(t0 + 1, 1)
            drain_echo(0)

            @pl.when(i < NTPAIR - 1)
            def _():
                issue_loads(t0 + 2, 0)

            compute(1)
            return carry

        lax.fori_loop(0, NTPAIR, pair, 0)
        drain_echo(1)

        pltpu.sync_copy(acc, out_hbm.at[c * NS + s])

    return k(xe4, ei4)


RB = 1024


def _tc_mlp(x_node, agg_t, W, b2):
    def mm(x_ref, p_ref, w_ref, b_ref, o_ref):
        w = w_ref[...]
        p = p_ref[...]
        a_t = p[:NS, :] + p[NS:, :]
        acc = jnp.dot(x_ref[...], w[:D_NODE, :], preferred_element_type=jnp.float32)
        acc += lax.dot_general(
            a_t, w[D_NODE:, :],
            dimension_numbers=(((0,), (0,)), ((), ())),
            preferred_element_type=jnp.float32,
        )
        o_ref[...] = acc + b_ref[...]

    nb = N_COLS // RB
    return pl.pallas_call(
        mm,
        grid=(nb,),
        in_specs=[
            pl.BlockSpec((RB, D_NODE), lambda i: (i, 0)),
            pl.BlockSpec((NC * NS, RB), lambda i: (0, i)),
            pl.BlockSpec((D_NODE + D_EDGE, D_OUT), lambda i: (0, 0)),
            pl.BlockSpec((1, D_OUT), lambda i: (0, 0)),
        ],
        out_specs=pl.BlockSpec((RB, D_OUT), lambda i: (i, 0)),
        out_shape=jax.ShapeDtypeStruct((N_NODES, D_OUT), jnp.float32),
    )(x_node, agg_t, W, b2)


def kernel(x_node, x_edge, edge_index, W, b):
    xe4 = x_edge.T.reshape(NC, FB, EB, 128).transpose(0, 2, 1, 3)
    ei4 = edge_index.T.reshape(2, EB, 128).transpose(1, 0, 2)
    agg_t, xe4_out, ei4_out = _sc_scatter_t(xe4, ei4)
    out = _tc_mlp(x_node, agg_t, W, b.reshape(1, D_OUT))
    x_edge_out = xe4_out.transpose(0, 2, 1, 3).reshape(D_EDGE, N_EDGES).T
    edge_index_out = ei4_out.transpose(1, 0, 2).reshape(2, N_EDGES).T
    return (out, x_edge_out, edge_index_out)


# speedup vs baseline: 1.5704x; 1.2304x over previous
import functools

import jax
import jax.numpy as jnp
from jax import lax
from jax.experimental import pallas as pl
from jax.experimental.pallas import tpu as pltpu
from jax.experimental.pallas import tpu_sc as plsc

N_NODES = 100000
N_EDGES = 3200000
D_EDGE = 16
D_NODE = 128
D_OUT = 128

NC = 2
NS = 16
LANES = 16
FB = 8
EB = N_EDGES // 128
EBC = EB // NC

CB = 25
CHUNK = CB * 128
NT = EBC // CB
NTPAIR = NT // 2
UNROLL = 5
N_COLS = 100352


def _sc_scatter_t(xe4, ei4):
    mesh = plsc.VectorSubcoreMesh(core_axis_name="c", subcore_axis_name="s")

    @functools.partial(
        pl.kernel,
        out_type=(
            jax.ShapeDtypeStruct((NC * NS, N_COLS), jnp.float32),
            jax.ShapeDtypeStruct((NC, EB, FB, 128), jnp.float32),
            jax.ShapeDtypeStruct((EB, 2, 128), jnp.int32),
        ),
        mesh=mesh,
        compiler_params=pltpu.CompilerParams(
            use_tc_tiling_on_sc=False, needs_layout_passes=False),
        scratch_types=[
            pltpu.VMEM((N_COLS,), jnp.float32),
            pltpu.VMEM((CB, 128), jnp.float32),
            pltpu.VMEM((CB, 128), jnp.float32),
            pltpu.VMEM((CB, 128), jnp.float32),
            pltpu.VMEM((CB, 2, 128), jnp.int32),
            pltpu.VMEM((CB, 2, 128), jnp.int32),
            pltpu.VMEM((CB, 2, 128), jnp.int32),
            pltpu.SemaphoreType.DMA,
            pltpu.SemaphoreType.DMA,
            pltpu.SemaphoreType.DMA,
            pltpu.SemaphoreType.DMA,
            pltpu.SemaphoreType.DMA,
            pltpu.SemaphoreType.DMA,
        ],
    )
    def k(xe_hbm, ei_hbm, out_hbm, oxe_hbm, oei_hbm, acc,
          xb0, xb1, xb2, ib0, ib1, ib2, sl0, sl1, sl2, se0, se1, se2):
        c = lax.axis_index("c")
        s = lax.axis_index("s")
        fb = s // FB
        fr = s % FB

        xb = (xb0, xb1, xb2)
        ib = (ib0, ib1, ib2)
        sl = (sl0, sl1, sl2)
        se = (se0, se1, se2)

        zero = jnp.zeros((LANES,), jnp.float32)

        def zb(i, carry):
            acc[pl.ds(i * (4 * LANES), LANES)] = zero
            acc[pl.ds(i * (4 * LANES) + LANES, LANES)] = zero
            acc[pl.ds(i * (4 * LANES) + 2 * LANES, LANES)] = zero
            acc[pl.ds(i * (4 * LANES) + 3 * LANES, LANES)] = zero
            return carry

        lax.fori_loop(0, N_COLS // (4 * LANES), zb, 0)

        jcore = c * EBC

        def issue_loads(t, b):
            jbase = jcore + t * CB
            pltpu.async_copy(
                xe_hbm.at[fb, pl.ds(jbase, CB), fr, :], xb[b], sl[b])
            pltpu.async_copy(ei_hbm.at[pl.ds(jbase, CB)], ib[b], sl[b])

        def drain_loads(b):
            pltpu.make_async_copy(
                xe_hbm.at[0, pl.ds(0, CB), 0, :], xb[b], sl[b]).wait()
            pltpu.make_async_copy(
                ei_hbm.at[pl.ds(0, CB)], ib[b], sl[b]).wait()

        def issue_echo(t, b):
            jbase = jcore + t * CB
            pltpu.async_copy(
                xb[b], oxe_hbm.at[fb, pl.ds(jbase, CB), fr, :], se[b])

            @pl.when(s == 0)
            def _():
                pltpu.async_copy(ib[b], oei_hbm.at[pl.ds(jbase, CB)], se[b])

        def drain_echo(b):
            pltpu.make_async_copy(
                xb[b], oxe_hbm.at[0, pl.ds(0, CB), 0, :], se[b]).wait()

            @pl.when(s == 0)
            def _():
                pltpu.make_async_copy(
                    ib[b], oei_hbm.at[pl.ds(0, CB)], se[b]).wait()

        def compute(b):
            @plsc.parallel_loop(0, CB, 1, unroll=UNROLL)
            def grp(jj):
                for u in range(FB):
                    sl16 = pl.ds(u * LANES, LANES)
                    v = xb[b][jj, sl16]
                    i0 = ib[b][jj, 0, sl16]
                    i1 = ib[b][jj, 1, sl16]
                    plsc.addupdate_scatter(acc, [i0], v)
                    plsc.addupdate_scatter(acc, [i1], v)

        issue_loads(0, 0)
        issue_loads(1, 1)

        def triple(j, carry):
            for u in range(3):
                t = 3 * j + u
                bc = u
                bn = (u + 2) % 3
                drain_loads(bc)
                issue_echo(t, bc)

                @pl.when(t > 0)
                def _():
                    drain_echo(bn)

                issue_loads(t + 2, bn)
                compute(bc)
            return carry

        lax.fori_loop(0, NT // 3, triple, 0)
        drain_loads(0)
        issue_echo(NT - 2, 0)
        drain_echo(2)
        compute(0)
        drain_loads(1)
        issue_echo(NT - 1, 1)
        compute(1)
        drain_echo(0)
        drain_echo(1)

        pltpu.sync_copy(acc, out_hbm.at[c * NS + s])

    return k(xe4, ei4)


RB = 1024


def _tc_mlp(x_node, agg_t, W, b2):
    def mm(x_ref, p_ref, w_ref, b_ref, o_ref):
        w = w_ref[...]
        p = p_ref[...]
        a_t = p[:NS, :] + p[NS:, :]
        acc = jnp.dot(x_ref[...], w[:D_NODE, :], preferred_element_type=jnp.float32)
        acc += lax.dot_general(
            a_t, w[D_NODE:, :],
            dimension_numbers=(((0,), (0,)), ((), ())),
            preferred_element_type=jnp.float32,
        )
        o_ref[...] = acc + b_ref[...]

    nb = N_COLS // RB
    return pl.pallas_call(
        mm,
        grid=(nb,),
        in_specs=[
            pl.BlockSpec((RB, D_NODE), lambda i: (i, 0)),
            pl.BlockSpec((NC * NS, RB), lambda i: (0, i)),
            pl.BlockSpec((D_NODE + D_EDGE, D_OUT), lambda i: (0, 0)),
            pl.BlockSpec((1, D_OUT), lambda i: (0, 0)),
        ],
        out_specs=pl.BlockSpec((RB, D_OUT), lambda i: (i, 0)),
        out_shape=jax.ShapeDtypeStruct((N_NODES, D_OUT), jnp.float32),
    )(x_node, agg_t, W, b2)


def kernel(x_node, x_edge, edge_index, W, b):
    xe4 = x_edge.T.reshape(NC, FB, EB, 128).transpose(0, 2, 1, 3)
    ei4 = edge_index.T.reshape(2, EB, 128).transpose(1, 0, 2)
    agg_t, xe4_out, ei4_out = _sc_scatter_t(xe4, ei4)
    out = _tc_mlp(x_node, agg_t, W, b.reshape(1, D_OUT))
    x_edge_out = xe4_out.transpose(0, 2, 1, 3).reshape(D_EDGE, N_EDGES).T
    edge_index_out = ei4_out.transpose(1, 0, 2).reshape(2, N_EDGES).T
    return (out, x_edge_out, edge_index_out)
